# baseline (speedup 1.0000x reference)
import jax
import jax.numpy as jnp
from jax import lax
from jax.experimental import pallas as pl
from jax.experimental.pallas import tpu as pltpu
from jax.experimental.pallas import tpu_sc as plsc

VOCAB = 1000000
DIM = 64
ROWS = 16384
COLS = 50

_INFO = plsc.get_sparse_core_info()
NC = _INFO.num_cores
NS = _INFO.num_subcores
NW = NC * NS

NGRP = VOCAB // 128
NPAIR = NGRP // 2
T2_ROWS = VOCAB // 2
TAIL_T2 = NGRP * 64

BLK = 128
NBLK = ROWS * COLS // BLK
BLK_PER_W = NBLK // NW
RB = ROWS // BLK


def _iota16():
    return lax.iota(jnp.int32, 16)


def _transpose_pairs(buf, outb):
    rows = [_iota16() + 16 * q for q in range(4)]

    @pl.loop(0, 64, unroll=4)
    def _(m):
        ca = jnp.full((16,), 2 * m, jnp.int32)
        cb = ca + 1
        for q in range(4):
            outb[m, pl.ds(16 * q, 16)] = plsc.load_gather(buf, [rows[q], ca])
        for q in range(4):
            outb[m, pl.ds(64 + 16 * q, 16)] = plsc.load_gather(buf, [rows[q], cb])


def _stage1_body(tT_hbm, tail_hbm, t2_hbm, buf_v, out_v, isems, osems):
    wid = lax.axis_index("s") * NC + lax.axis_index("c")
    nq = NGRP // 4
    q0 = wid * (nq // NW) + jnp.minimum(wid, nq % NW)
    nquad = (nq // NW) + jnp.where(wid < nq % NW, 1, 0)
    g0 = 4 * q0
    ng = 4 * nquad

    def in_fire(n, b):
        for m in range(8):
            pltpu.async_copy(
                tT_hbm.at[pl.ds(8 * m, 8), pl.ds((g0 + n) * 128, 128)],
                buf_v.at[b, pl.ds(8 * m, 8), pl.ds(0, 128)], isems.at[b])

    def in_wait(b):
        pltpu.make_async_copy(tT_hbm.at[:, pl.ds(0, 128)],
                              buf_v.at[0, :, pl.ds(0, 128)],
                              isems.at[b]).wait()

    def out_fire(n, b):
        pltpu.async_copy(out_v.at[b], t2_hbm.at[pl.ds((g0 + n) * 64, 64)],
                         osems.at[b])

    def out_wait(b):
        pltpu.make_async_copy(out_v.at[0], t2_hbm.at[pl.ds(0, 64)],
                              osems.at[b]).wait()

    for b in range(4):
        in_fire(b, b)

    @pl.loop(0, nquad)
    def _(qk):
        n0 = 4 * qk
        for k in range(4):
            n = n0 + k
            in_wait(k)

            @pl.when(qk > 0)
            def _():
                out_wait(k)
            _transpose_pairs(buf_v.at[k], out_v.at[k])
            out_fire(n, k)

            @pl.when(n + 4 < ng)
            def _():
                in_fire(n + 4, k)

    for k in range(4):
        out_wait(k)

    @pl.when(wid == 0)
    def _():
        pltpu.sync_copy(tail_hbm, buf_v.at[0, pl.ds(0, 32), pl.ds(0, 128)])
        pltpu.sync_copy(buf_v.at[0, pl.ds(0, 32), pl.ds(0, 128)],
                        t2_hbm.at[pl.ds(TAIL_T2, 32)])


def _prep_block(idxr, idx2, colb):
    for s in range(8):
        v = idxr[pl.ds(16 * s, 16)]
        idx2[pl.ds(16 * s, 16)] = lax.shift_right_logical(v, 1)
        colb[pl.ds(16 * s, 16)] = lax.shift_left(jnp.bitwise_and(v, 1), 6)


def _transpose_select(g_v, colb, outb):
    rows = [_iota16() + 16 * q for q in range(8)]
    cbs = [colb[pl.ds(16 * q, 16)] for q in range(8)]

    @pl.loop(0, 64, unroll=2)
    def _(d):
        for q in range(8):
            outb[d, pl.ds(16 * q, 16)] = plsc.load_gather(
                g_v, [rows[q], cbs[q] + d])


def _stage2_body(t2_hbm, xT_hbm, out_hbm, idxr_v, idx2_v, colb_v, g_v,
                 out_v, xsems, gsems, osems):
    wid = lax.axis_index("s") * NC + lax.axis_index("c")
    t0 = wid * BLK_PER_W
    D = 5

    def x_fire(t, b):
        c = t // RB
        r0 = (t % RB) * BLK
        pltpu.async_copy(xT_hbm.at[c, pl.ds(r0, BLK)], idxr_v.at[b],
                         xsems.at[b])

    def x_wait(b):
        pltpu.make_async_copy(xT_hbm.at[0, pl.ds(0, BLK)], idxr_v.at[b],
                              xsems.at[b]).wait()

    def g_fire(b):
        pltpu.async_copy(t2_hbm.at[idx2_v.at[b]],
                         g_v.at[b, :, pl.ds(0, 128)], gsems.at[b])

    def g_wait(b):
        pltpu.make_async_copy(t2_hbm.at[idx2_v.at[b]],
                              g_v.at[0, :, pl.ds(0, 128)],
                              gsems.at[b]).wait()

    def o_fire(t, b):
        c = t // RB
        r0 = (t % RB) * BLK
        for m in range(8):
            pltpu.async_copy(
                out_v.at[b, pl.ds(8 * m, 8), :],
                out_hbm.at[c, pl.ds(8 * m, 8), pl.ds(r0, BLK)], osems.at[b])

    def o_wait(b):
        pltpu.make_async_copy(out_v.at[0], out_hbm.at[0, :, pl.ds(0, BLK)],
                              osems.at[b]).wait()

    def stage_up(t, b):
        x_wait(b)
        _prep_block(idxr_v.at[b], idx2_v.at[b], colb_v.at[b])
        g_fire(b)

    def retire(t, b, ob, wait_out):
        g_wait(b)
        if wait_out:
            o_wait(ob)
        _transpose_select(g_v.at[b], colb_v.at[b], out_v.at[ob])
        o_fire(t, ob)

    for b in range(D):
        x_fire(t0 + b, b)
    for b in range(D - 1):
        stage_up(t0 + b, b)

    for k in range(D):
        tt = t0 + k
        retire(tt, k, k % 2, k >= 2)
        x_fire(tt + D, k)
        stage_up(tt + D - 1, (k + D - 1) % D)

    @pl.loop(0, (BLK_PER_W - 2 * D) // (2 * D))
    def _(pd):
        tb = t0 + D + 2 * D * pd
        for k in range(2 * D):
            tt = tb + k
            retire(tt, (D + k) % D, (D + k) % 2, True)
            x_fire(tt + D, (D + k) % D)
            stage_up(tt + D - 1, (D + k - 1) % D)

    tb = t0 + BLK_PER_W - D
    retire(tb, tb % D if False else 0, (D + 0) % 2, True)
    stage_up(tb + D - 1, D - 1)
    for k in range(1, D):
        retire(tb + k, k, (D + k) % 2, True)
    o_wait(0)
    o_wait(1)


_MESH = dict(core_axis_name="c", subcore_axis_name="s")


@jax.jit
def _sc_embed(tT, tail, xT):
    s1 = pl.kernel(
        _stage1_body,
        out_type=jax.ShapeDtypeStruct((T2_ROWS, 128), jnp.float32),
        mesh=plsc.VectorSubcoreMesh(**_MESH),
        scratch_types=[
            pltpu.VMEM((4, 64, 128), jnp.float32),
            pltpu.VMEM((4, 64, 128), jnp.float32),
            pltpu.SemaphoreType.DMA((4,)),
            pltpu.SemaphoreType.DMA((4,)),
        ],
        compiler_params=pltpu.CompilerParams(use_tc_tiling_on_sc=True,
                                             needs_layout_passes=False),
    )
    t2 = s1(tT, tail)
    s2 = pl.kernel(
        _stage2_body,
        out_type=jax.ShapeDtypeStruct((COLS, DIM, ROWS), jnp.float32),
        mesh=plsc.VectorSubcoreMesh(**_MESH),
        scratch_types=[
            pltpu.VMEM((5, BLK), jnp.int32),
            pltpu.VMEM((5, BLK), jnp.int32),
            pltpu.VMEM((5, BLK), jnp.int32),
            pltpu.VMEM((5, BLK, 128), jnp.float32),
            pltpu.VMEM((2, DIM, BLK), jnp.float32),
            pltpu.SemaphoreType.DMA((5,)),
            pltpu.SemaphoreType.DMA((5,)),
            pltpu.SemaphoreType.DMA((2,)),
        ],
        compiler_params=pltpu.CompilerParams(use_tc_tiling_on_sc=True,
                                             needs_layout_passes=False),
    )
    return s2(t2, xT)


def kernel(x, table):
    tT = table.T
    tail = table[NGRP * 128:].reshape(32, 128)
    xT = x.astype(jnp.int32).T
    out_phys = _sc_embed(tT, tail, xT)
    return out_phys.transpose(2, 0, 1)


# speedup vs baseline: 2.2007x; 2.2007x over previous
import jax
import jax.numpy as jnp
from jax import lax
from jax.experimental import pallas as pl
from jax.experimental.pallas import tpu as pltpu
from jax.experimental.pallas import tpu_sc as plsc

VOCAB = 1000000
DIM = 64
ROWS = 16384
COLS = 50
B = ROWS * COLS

_INFO = plsc.get_sparse_core_info()
NC = _INFO.num_cores
NS = _INFO.num_subcores
NW = NC * NS
B_PER_W = B // NW

CH = 128
N_CHUNKS = B_PER_W // CH
NBUF = 8
GDEPTH = 6


def _body(x_hbm, table_hbm, out_hbm, idx_v, rows_v, gsems, ssems):
    wid = lax.axis_index("s") * NC + lax.axis_index("c")
    base = wid * B_PER_W

    pltpu.sync_copy(x_hbm.at[wid], idx_v)

    def g_fire(j, b):
        pltpu.async_copy(table_hbm.at[idx_v.at[j]], rows_v.at[b], gsems.at[b])

    def g_wait(j, b):
        pltpu.make_async_copy(table_hbm.at[idx_v.at[j]], rows_v.at[b],
                              gsems.at[b]).wait()

    def s_fire(j, b):
        pltpu.async_copy(rows_v.at[b], out_hbm.at[pl.ds(base + j * CH, CH)],
                         ssems.at[b])

    def s_wait(j, b):
        pltpu.make_async_copy(rows_v.at[b],
                              out_hbm.at[pl.ds(base + j * CH, CH)],
                              ssems.at[b]).wait()

    for j in range(GDEPTH):
        g_fire(j, j % NBUF)

    for j in range(NBUF):
        g_wait(j, j % NBUF)
        s_fire(j, j % NBUF)
        if j >= NBUF - GDEPTH:
            s_wait(j - (NBUF - GDEPTH), (j - (NBUF - GDEPTH)) % NBUF)
        g_fire(j + GDEPTH, (j + GDEPTH) % NBUF)

    @pl.loop(NBUF, N_CHUNKS - NBUF, step=NBUF)
    def _(g):
        for b in range(NBUF):
            j = g + b
            g_wait(j, b)
            s_fire(j, b)
            s_wait(j - (NBUF - GDEPTH), (b - (NBUF - GDEPTH)) % NBUF)
            g_fire(j + GDEPTH, (b + GDEPTH) % NBUF)

    for j in range(N_CHUNKS - NBUF, N_CHUNKS):
        g_wait(j, j % NBUF)
        s_fire(j, j % NBUF)
        s_wait(j - (NBUF - GDEPTH), (j - (NBUF - GDEPTH)) % NBUF)
        if j + GDEPTH < N_CHUNKS:
            g_fire(j + GDEPTH, (j + GDEPTH) % NBUF)

    for j in range(N_CHUNKS - (NBUF - GDEPTH), N_CHUNKS):
        s_wait(j, j % NBUF)


@jax.jit
def _sc_gather(x3, table):
    k = pl.kernel(
        _body,
        out_type=jax.ShapeDtypeStruct((B, DIM), jnp.float32),
        mesh=plsc.VectorSubcoreMesh(core_axis_name="c", subcore_axis_name="s"),
        scratch_types=[
            pltpu.VMEM((N_CHUNKS, CH), jnp.int32),
            pltpu.VMEM((NBUF, CH, DIM), jnp.float32),
            pltpu.SemaphoreType.DMA((NBUF,)),
            pltpu.SemaphoreType.DMA((NBUF,)),
        ],
        compiler_params=pltpu.CompilerParams(use_tc_tiling_on_sc=False),
    )
    return k(x3, table)


def kernel(x, table):
    x3 = x.reshape(NW, N_CHUNKS, CH).astype(jnp.int32)
    out = _sc_gather(x3, table)
    return out.reshape(ROWS, COLS, DIM)
